# Initial kernel scaffold; baseline (speedup 1.0000x reference)
import functools

import jax
import jax.numpy as jnp
from jax import lax
from jax.experimental import pallas as pl
from jax.experimental.pallas import tpu as pltpu
from jax.experimental.pallas import tpu_sc as plsc

NC = 2
NS = 16
CHUNK = 80


def _encode_body(x_ref, w_ref, b_ref, o_ref):
    h = jnp.dot(x_ref[...], w_ref[...], preferred_element_type=jnp.float32)
    o_ref[...] = jnp.maximum(h + b_ref[...], 0.0)


def _encode(x, w, b):
    return pl.pallas_call(
        _encode_body,
        out_shape=jax.ShapeDtypeStruct(x.shape, jnp.float32),
    )(x, w, b)


def _edge_body(ea_ref, w_ref, b_ref, o_ref):
    e = jnp.dot(ea_ref[...], w_ref[...], preferred_element_type=jnp.float32)
    o_ref[...] = e + b_ref[...]


def _edge_transform(edge_attr, w, b):
    E, ED = edge_attr.shape
    D = w.shape[1]
    BE = 6400
    grid = E // BE
    return pl.pallas_call(
        _edge_body,
        grid=(grid,),
        in_specs=[
            pl.BlockSpec((BE, ED), lambda i: (i, 0)),
            pl.BlockSpec((ED, D), lambda i: (0, 0)),
            pl.BlockSpec((1, D), lambda i: (0, 0)),
        ],
        out_specs=pl.BlockSpec((BE, D), lambda i: (i, 0)),
        out_shape=jax.ShapeDtypeStruct((E, D), jnp.float32),
    )(edge_attr, w, b)


def _post_body(h_ref, p0_ref, p1_ref, w1_ref, b1_ref, w2_ref, b2_ref,
               g_ref, be_ref, o_ref):
    h = h_ref[...]
    z = h + p0_ref[...] + p1_ref[...]
    u = jnp.dot(z, w1_ref[...], preferred_element_type=jnp.float32)
    u = jnp.maximum(u + b1_ref[...], 0.0)
    v = jnp.dot(u, w2_ref[...], preferred_element_type=jnp.float32)
    v = v + b2_ref[...]
    mu = jnp.mean(v, axis=0, keepdims=True)
    var = jnp.mean((v - mu) * (v - mu), axis=0, keepdims=True)
    zn = (v - mu) / jnp.sqrt(var + 1e-5) * g_ref[...] + be_ref[...]
    o_ref[...] = jnp.maximum(zn, 0.0) + h


def _post(h, p0, p1, w1, b1, w2, b2, g, be):
    return pl.pallas_call(
        _post_body,
        out_shape=jax.ShapeDtypeStruct(h.shape, jnp.float32),
    )(h, p0, p1, w1, b1, w2, b2, g, be)


def _pool_head_body(h_ref, batch_ref, wh1_ref, bh1_ref, wh2_ref, bh2_ref,
                    wh3_ref, bh3_ref, o_ref, *, num_groups):
    h = h_ref[...]
    n = h.shape[0]
    b = batch_ref[...]
    gid = lax.broadcasted_iota(jnp.int32, (n, num_groups), 1)
    onehot = (b == gid).astype(jnp.float32)
    dnums = (((0,), (0,)), ((), ()))
    s = lax.dot_general(onehot, h, dnums, preferred_element_type=jnp.float32)
    ones = jnp.ones((n, 1), jnp.float32)
    cnt = lax.dot_general(onehot, ones, dnums,
                          preferred_element_type=jnp.float32)
    mean = s / jnp.maximum(cnt, 1.0)
    rows = []
    for g in range(num_groups):
        m = b == g
        hm = jnp.where(m, h, -jnp.inf)
        rows.append(jnp.max(hm, axis=0, keepdims=True))
    mx = jnp.concatenate(rows, axis=0)
    mx = jnp.where(jnp.isfinite(mx), mx, 0.0)
    p = jnp.concatenate([mean, s, mx], axis=1)
    o = jnp.dot(p, wh1_ref[...], preferred_element_type=jnp.float32)
    o = jnp.maximum(o + bh1_ref[...], 0.0)
    o = jnp.dot(o, wh2_ref[...], preferred_element_type=jnp.float32)
    o = jnp.maximum(o + bh2_ref[...], 0.0)
    o = jnp.dot(o, wh3_ref[...], preferred_element_type=jnp.float32)
    o_ref[...] = o + bh3_ref[...]


def _pool_head(h, batch_col, wh1, bh1, wh2, bh2, wh3, bh3, num_groups):
    body = functools.partial(_pool_head_body, num_groups=num_groups)
    return pl.pallas_call(
        body,
        out_shape=jax.ShapeDtypeStruct((num_groups, 1), jnp.float32),
    )(h, batch_col, wh1, bh1, wh2, bh2, wh3, bh3)


def _sc_agg_body(h_hbm, e_hbm, src_hbm, dst_hbm, out_hbm,
                 src_v, dst_v, rows_v, e_v, m_v, zbuf, agg_sh, sem,
                 *, n_nodes, edges_per_worker, n_chunks, d):
    c = lax.axis_index("c")
    s = lax.axis_index("s")
    wid = s * NC + c
    nvec = d // 16
    zrows = zbuf.shape[0]
    rows_per_sub = n_nodes // NS

    def zero_row(i, carry):
        for j in range(nvec):
            zbuf[i, pl.ds(j * 16, 16)] = jnp.zeros((16,), jnp.float32)
        return carry

    lax.fori_loop(0, zrows, zero_row, 0)
    for r in range(rows_per_sub // zrows):
        pltpu.sync_copy(zbuf, agg_sh.at[pl.ds(s * rows_per_sub + r * zrows,
                                              zrows)])
    plsc.subcore_barrier()

    base0 = wid * edges_per_worker

    def chunk(k, carry):
        base = base0 + k * CHUNK
        pltpu.sync_copy(src_hbm.at[pl.ds(base, CHUNK)], src_v)
        pltpu.sync_copy(dst_hbm.at[pl.ds(base, CHUNK)], dst_v)
        pltpu.async_copy(h_hbm.at[src_v], rows_v, sem).wait()
        pltpu.sync_copy(e_hbm.at[pl.ds(base, CHUNK)], e_v)

        def row(r, inner):
            for j in range(nvec):
                sl = pl.ds(j * 16, 16)
                m_v[r, sl] = jnp.maximum(rows_v[r, sl] + e_v[r, sl], 0.0)
            return inner

        lax.fori_loop(0, CHUNK, row, 0)
        pltpu.sync_copy(m_v, agg_sh.at[dst_v], add=True)
        return carry

    lax.fori_loop(0, n_chunks, chunk, 0)
    plsc.subcore_barrier()

    for r in range(rows_per_sub // zrows):
        off = s * rows_per_sub + r * zrows
        pltpu.sync_copy(agg_sh.at[pl.ds(off, zrows)],
                        out_hbm.at[c, pl.ds(off, zrows)])


def _sc_agg(h, e, src, dst):
    n_nodes, d = h.shape
    n_edges = e.shape[0]
    epw = n_edges // (NC * NS)
    n_chunks = epw // CHUNK
    zrows = 125
    body = functools.partial(
        _sc_agg_body, n_nodes=n_nodes, edges_per_worker=epw,
        n_chunks=n_chunks, d=d)
    mesh = plsc.VectorSubcoreMesh(core_axis_name="c", subcore_axis_name="s")
    f = pl.kernel(
        body,
        out_type=jax.ShapeDtypeStruct((NC, n_nodes, d), jnp.float32),
        mesh=mesh,
        scratch_types=[
            pltpu.VMEM((CHUNK,), jnp.int32),
            pltpu.VMEM((CHUNK,), jnp.int32),
            pltpu.VMEM((CHUNK, d), jnp.float32),
            pltpu.VMEM((CHUNK, d), jnp.float32),
            pltpu.VMEM((CHUNK, d), jnp.float32),
            pltpu.VMEM((zrows, d), jnp.float32),
            pltpu.VMEM_SHARED((n_nodes, d), jnp.float32),
            pltpu.SemaphoreType.DMA,
        ],
    )
    return f(h, e, src, dst)


def kernel(x, edge_index, edge_attr, batch, Wenc, benc, Wedge, bedge,
           W1, b1, W2, b2, gamma, beta, Wh1, bh1, Wh2, bh2, Wh3, bh3):
    num_layers = Wedge.shape[0]
    num_groups = 64
    src = edge_index[0]
    dst = edge_index[1]
    h = _encode(x, Wenc, benc.reshape(1, -1))
    for i in range(num_layers):
        e = _edge_transform(edge_attr, Wedge[i], bedge[i].reshape(1, -1))
        parts = _sc_agg(h, e, src, dst)
        h = _post(h, parts[0], parts[1], W1[i], b1[i].reshape(1, -1),
                  W2[i], b2[i].reshape(1, -1), gamma[i].reshape(1, -1),
                  beta[i].reshape(1, -1))
    out = _pool_head(h, batch.reshape(-1, 1), Wh1, bh1.reshape(1, -1),
                     Wh2, bh2.reshape(1, -1), Wh3, bh3.reshape(1, -1),
                     num_groups)
    return out.reshape(-1)


# speedup vs baseline: 2.4656x; 2.4656x over previous
import functools

import jax
import jax.numpy as jnp
from jax import lax
from jax.experimental import pallas as pl
from jax.experimental.pallas import tpu as pltpu
from jax.experimental.pallas import tpu_sc as plsc

NC = 2
NS = 16
CHUNK = 80


def _encode_body(x_ref, w_ref, b_ref, o_ref):
    h = jnp.dot(x_ref[...], w_ref[...], preferred_element_type=jnp.float32)
    o_ref[...] = jnp.maximum(h + b_ref[...], 0.0)


def _encode(x, w, b):
    return pl.pallas_call(
        _encode_body,
        out_shape=jax.ShapeDtypeStruct(x.shape, jnp.float32),
    )(x, w, b)


def _edge_body(ea_ref, w_ref, b_ref, o_ref):
    e = jnp.dot(ea_ref[...], w_ref[...], preferred_element_type=jnp.float32)
    o_ref[...] = e + b_ref[...]


def _edge_transform(edge_attr, w, b):
    E, ED = edge_attr.shape
    D = w.shape[1]
    BE = 6400
    grid = E // BE
    return pl.pallas_call(
        _edge_body,
        grid=(grid,),
        in_specs=[
            pl.BlockSpec((BE, ED), lambda i: (i, 0)),
            pl.BlockSpec((ED, D), lambda i: (0, 0)),
            pl.BlockSpec((1, D), lambda i: (0, 0)),
        ],
        out_specs=pl.BlockSpec((BE, D), lambda i: (i, 0)),
        out_shape=jax.ShapeDtypeStruct((E, D), jnp.float32),
    )(edge_attr, w, b)


def _post_body(h_ref, p0_ref, p1_ref, w1_ref, b1_ref, w2_ref, b2_ref,
               g_ref, be_ref, o_ref):
    h = h_ref[...]
    z = h + p0_ref[...] + p1_ref[...]
    u = jnp.dot(z, w1_ref[...], preferred_element_type=jnp.float32)
    u = jnp.maximum(u + b1_ref[...], 0.0)
    v = jnp.dot(u, w2_ref[...], preferred_element_type=jnp.float32)
    v = v + b2_ref[...]
    mu = jnp.mean(v, axis=0, keepdims=True)
    var = jnp.mean((v - mu) * (v - mu), axis=0, keepdims=True)
    zn = (v - mu) / jnp.sqrt(var + 1e-5) * g_ref[...] + be_ref[...]
    o_ref[...] = jnp.maximum(zn, 0.0) + h


def _post(h, p0, p1, w1, b1, w2, b2, g, be):
    return pl.pallas_call(
        _post_body,
        out_shape=jax.ShapeDtypeStruct(h.shape, jnp.float32),
    )(h, p0, p1, w1, b1, w2, b2, g, be)


def _pool_head_body(h_ref, batch_ref, wh1_ref, bh1_ref, wh2_ref, bh2_ref,
                    wh3_ref, bh3_ref, o_ref, mx_ref, *, num_groups):
    i = pl.program_id(0)
    b = batch_ref[...]

    @pl.when(i < num_groups)
    def _():
        hm = jnp.where(b == i, h_ref[...], -jnp.inf)
        mx_ref[pl.ds(i, 1), :] = jnp.max(hm, axis=0, keepdims=True)

    @pl.when(i == num_groups)
    def _():
        h = h_ref[...]
        n = h.shape[0]
        gid = lax.broadcasted_iota(jnp.int32, (n, num_groups), 1)
        onehot = (b == gid).astype(jnp.float32)
        dnums = (((0,), (0,)), ((), ()))
        s = lax.dot_general(onehot, h, dnums,
                            preferred_element_type=jnp.float32)
        ones = jnp.ones((n, 1), jnp.float32)
        cnt = lax.dot_general(onehot, ones, dnums,
                              preferred_element_type=jnp.float32)
        mean = s / jnp.maximum(cnt, 1.0)
        mx = mx_ref[...]
        mx = jnp.where(jnp.isfinite(mx), mx, 0.0)
        p = jnp.concatenate([mean, s, mx], axis=1)
        o = jnp.dot(p, wh1_ref[...], preferred_element_type=jnp.float32)
        o = jnp.maximum(o + bh1_ref[...], 0.0)
        o = jnp.dot(o, wh2_ref[...], preferred_element_type=jnp.float32)
        o = jnp.maximum(o + bh2_ref[...], 0.0)
        o = jnp.dot(o, wh3_ref[...], preferred_element_type=jnp.float32)
        o_ref[...] = o + bh3_ref[...]


def _pool_head(h, batch_col, wh1, bh1, wh2, bh2, wh3, bh3, num_groups):
    body = functools.partial(_pool_head_body, num_groups=num_groups)
    n, d = h.shape
    const = lambda i: (0, 0)
    return pl.pallas_call(
        body,
        grid=(num_groups + 1,),
        in_specs=[
            pl.BlockSpec(h.shape, const),
            pl.BlockSpec(batch_col.shape, const),
            pl.BlockSpec(wh1.shape, const),
            pl.BlockSpec(bh1.shape, const),
            pl.BlockSpec(wh2.shape, const),
            pl.BlockSpec(bh2.shape, const),
            pl.BlockSpec(wh3.shape, const),
            pl.BlockSpec(bh3.shape, const),
        ],
        out_specs=pl.BlockSpec((num_groups, 1), const),
        out_shape=jax.ShapeDtypeStruct((num_groups, 1), jnp.float32),
        scratch_shapes=[pltpu.VMEM((num_groups, d), jnp.float32)],
    )(h, batch_col, wh1, bh1, wh2, bh2, wh3, bh3)


def _sc_agg_body(h_hbm, e_hbm, src_hbm, dst_hbm, out_hbm,
                 src_v, dst_v, rows_v, e_v, m_v, zbuf, agg_sh, sem,
                 *, n_nodes, edges_per_worker, n_chunks, d):
    c = lax.axis_index("c")
    s = lax.axis_index("s")
    wid = s * NC + c
    nvec = d // 16
    zrows = zbuf.shape[0]
    n_row_chunks = n_nodes // zrows
    tail = n_nodes - n_row_chunks * zrows
    per_sub = (n_row_chunks + NS - 1) // NS

    def zero_row(i, carry):
        for j in range(nvec):
            zbuf[i, pl.ds(j * 16, 16)] = jnp.zeros((16,), jnp.float32)
        return carry

    lax.fori_loop(0, zrows, zero_row, 0)
    for k in range(per_sub):
        cid = k * NS + s

        @pl.when(cid < n_row_chunks)
        def _():
            pltpu.sync_copy(zbuf, agg_sh.at[pl.ds(cid * zrows, zrows)])
    if tail:
        @pl.when(s == NS - 1)
        def _():
            pltpu.sync_copy(zbuf.at[pl.ds(0, tail)],
                            agg_sh.at[pl.ds(n_row_chunks * zrows, tail)])
    plsc.subcore_barrier()

    base0 = wid * edges_per_worker

    def chunk(k, carry):
        base = base0 + k * CHUNK
        pltpu.sync_copy(src_hbm.at[pl.ds(base, CHUNK)], src_v)
        pltpu.sync_copy(dst_hbm.at[pl.ds(base, CHUNK)], dst_v)
        pltpu.async_copy(h_hbm.at[src_v], rows_v, sem).wait()
        pltpu.sync_copy(e_hbm.at[pl.ds(base, CHUNK)], e_v)

        def row(r, inner):
            for j in range(nvec):
                sl = pl.ds(j * 16, 16)
                m_v[r, sl] = jnp.maximum(rows_v[r, sl] + e_v[r, sl], 0.0)
            return inner

        lax.fori_loop(0, CHUNK, row, 0)
        pltpu.sync_copy(m_v, agg_sh.at[dst_v], add=True)
        return carry

    lax.fori_loop(0, n_chunks, chunk, 0)
    plsc.subcore_barrier()

    for k in range(per_sub):
        cid = k * NS + s

        @pl.when(cid < n_row_chunks)
        def _():
            off = cid * zrows
            pltpu.sync_copy(agg_sh.at[pl.ds(off, zrows)],
                            out_hbm.at[c, pl.ds(off, zrows)])
    if tail:
        @pl.when(s == NS - 1)
        def _():
            off = n_row_chunks * zrows
            pltpu.sync_copy(agg_sh.at[pl.ds(off, tail)],
                            out_hbm.at[c, pl.ds(off, tail)])


def _sc_agg(h, e, src, dst):
    n_nodes, d = h.shape
    n_edges = e.shape[0]
    epw = n_edges // (NC * NS)
    n_chunks = epw // CHUNK
    zrows = 128
    body = functools.partial(
        _sc_agg_body, n_nodes=n_nodes, edges_per_worker=epw,
        n_chunks=n_chunks, d=d)
    mesh = plsc.VectorSubcoreMesh(core_axis_name="c", subcore_axis_name="s")
    f = pl.kernel(
        body,
        out_type=jax.ShapeDtypeStruct((NC, n_nodes, d), jnp.float32),
        mesh=mesh,
        scratch_types=[
            pltpu.VMEM((CHUNK,), jnp.int32),
            pltpu.VMEM((CHUNK,), jnp.int32),
            pltpu.VMEM((CHUNK, d), jnp.float32),
            pltpu.VMEM((CHUNK, d), jnp.float32),
            pltpu.VMEM((CHUNK, d), jnp.float32),
            pltpu.VMEM((zrows, d), jnp.float32),
            pltpu.VMEM_SHARED((n_nodes, d), jnp.float32),
            pltpu.SemaphoreType.DMA,
        ],
    )
    return f(h, e, src, dst)


def kernel(x, edge_index, edge_attr, batch, Wenc, benc, Wedge, bedge,
           W1, b1, W2, b2, gamma, beta, Wh1, bh1, Wh2, bh2, Wh3, bh3):
    num_layers = Wedge.shape[0]
    num_groups = 64
    src = edge_index[0]
    dst = edge_index[1]
    h = _encode(x, Wenc, benc.reshape(1, -1))
    for i in range(num_layers):
        e = _edge_transform(edge_attr, Wedge[i], bedge[i].reshape(1, -1))
        parts = _sc_agg(h, e, src, dst)
        h = _post(h, parts[0], parts[1], W1[i], b1[i].reshape(1, -1),
                  W2[i], b2[i].reshape(1, -1), gamma[i].reshape(1, -1),
                  beta[i].reshape(1, -1))
    out = _pool_head(h, batch.reshape(-1, 1), Wh1, bh1.reshape(1, -1),
                     Wh2, bh2.reshape(1, -1), Wh3, bh3.reshape(1, -1),
                     num_groups)
    return out.reshape(-1)


# speedup vs baseline: 4.8944x; 1.9850x over previous
import functools

import jax
import jax.numpy as jnp
from jax import lax
from jax.experimental import pallas as pl
from jax.experimental.pallas import tpu as pltpu
from jax.experimental.pallas import tpu_sc as plsc

NC = 2
NS = 16
CHUNK = 80


def _encode_body(x_ref, w_ref, b_ref, o_ref):
    h = jnp.dot(x_ref[...], w_ref[...], preferred_element_type=jnp.float32)
    o_ref[...] = jnp.maximum(h + b_ref[...], 0.0)


def _encode(x, w, b):
    return pl.pallas_call(
        _encode_body,
        out_shape=jax.ShapeDtypeStruct(x.shape, jnp.float32),
    )(x, w, b)


def _edge_body(ea_ref, w_ref, b_ref, o_ref):
    e = jnp.dot(ea_ref[...], w_ref[...], preferred_element_type=jnp.float32)
    o_ref[...] = e + b_ref[...]


def _edge_transform(edge_attr, w, b):
    E, ED = edge_attr.shape
    D = w.shape[1]
    BE = 6400
    grid = E // BE
    return pl.pallas_call(
        _edge_body,
        grid=(grid,),
        in_specs=[
            pl.BlockSpec((BE, ED), lambda i: (i, 0)),
            pl.BlockSpec((ED, D), lambda i: (0, 0)),
            pl.BlockSpec((1, D), lambda i: (0, 0)),
        ],
        out_specs=pl.BlockSpec((BE, D), lambda i: (i, 0)),
        out_shape=jax.ShapeDtypeStruct((E, D), jnp.float32),
    )(edge_attr, w, b)


def _post_body(h_ref, p0_ref, p1_ref, w1_ref, b1_ref, w2_ref, b2_ref,
               g_ref, be_ref, o_ref):
    h = h_ref[...]
    z = h + p0_ref[...] + p1_ref[...]
    u = jnp.dot(z, w1_ref[...], preferred_element_type=jnp.float32)
    u = jnp.maximum(u + b1_ref[...], 0.0)
    v = jnp.dot(u, w2_ref[...], preferred_element_type=jnp.float32)
    v = v + b2_ref[...]
    mu = jnp.mean(v, axis=0, keepdims=True)
    var = jnp.mean((v - mu) * (v - mu), axis=0, keepdims=True)
    zn = (v - mu) / jnp.sqrt(var + 1e-5) * g_ref[...] + be_ref[...]
    o_ref[...] = jnp.maximum(zn, 0.0) + h


def _post(h, p0, p1, w1, b1, w2, b2, g, be):
    return pl.pallas_call(
        _post_body,
        out_shape=jax.ShapeDtypeStruct(h.shape, jnp.float32),
    )(h, p0, p1, w1, b1, w2, b2, g, be)


def _pool_head_body(h_ref, batch_ref, wh1_ref, bh1_ref, wh2_ref, bh2_ref,
                    wh3_ref, bh3_ref, o_ref, mx_ref, *, num_groups):
    i = pl.program_id(0)
    b = batch_ref[...]

    @pl.when(i < num_groups)
    def _():
        hm = jnp.where(b == i, h_ref[...], -jnp.inf)
        mx_ref[pl.ds(i, 1), :] = jnp.max(hm, axis=0, keepdims=True)

    @pl.when(i == num_groups)
    def _():
        h = h_ref[...]
        n = h.shape[0]
        gid = lax.broadcasted_iota(jnp.int32, (n, num_groups), 1)
        onehot = (b == gid).astype(jnp.float32)
        dnums = (((0,), (0,)), ((), ()))
        s = lax.dot_general(onehot, h, dnums,
                            preferred_element_type=jnp.float32)
        ones = jnp.ones((n, 1), jnp.float32)
        cnt = lax.dot_general(onehot, ones, dnums,
                              preferred_element_type=jnp.float32)
        mean = s / jnp.maximum(cnt, 1.0)
        mx = mx_ref[...]
        mx = jnp.where(jnp.isfinite(mx), mx, 0.0)
        p = jnp.concatenate([mean, s, mx], axis=1)
        o = jnp.dot(p, wh1_ref[...], preferred_element_type=jnp.float32)
        o = jnp.maximum(o + bh1_ref[...], 0.0)
        o = jnp.dot(o, wh2_ref[...], preferred_element_type=jnp.float32)
        o = jnp.maximum(o + bh2_ref[...], 0.0)
        o = jnp.dot(o, wh3_ref[...], preferred_element_type=jnp.float32)
        o_ref[...] = o + bh3_ref[...]


def _pool_head(h, batch_col, wh1, bh1, wh2, bh2, wh3, bh3, num_groups):
    body = functools.partial(_pool_head_body, num_groups=num_groups)
    n, d = h.shape
    const = lambda i: (0, 0)
    return pl.pallas_call(
        body,
        grid=(num_groups + 1,),
        in_specs=[
            pl.BlockSpec(h.shape, const),
            pl.BlockSpec(batch_col.shape, const),
            pl.BlockSpec(wh1.shape, const),
            pl.BlockSpec(bh1.shape, const),
            pl.BlockSpec(wh2.shape, const),
            pl.BlockSpec(bh2.shape, const),
            pl.BlockSpec(wh3.shape, const),
            pl.BlockSpec(bh3.shape, const),
        ],
        out_specs=pl.BlockSpec((num_groups, 1), const),
        out_shape=jax.ShapeDtypeStruct((num_groups, 1), jnp.float32),
        scratch_shapes=[pltpu.VMEM((num_groups, d), jnp.float32)],
    )(h, batch_col, wh1, bh1, wh2, bh2, wh3, bh3)


def _sc_agg_body(h_hbm, e_hbm, src_hbm, dst_hbm, out_hbm,
                 src0, dst0, m0, src1, dst1, m1, src2, dst2, m2,
                 zbuf, agg_sh,
                 semi0, semd0, seme0, semg0, sems0,
                 semi1, semd1, seme1, semg1, sems1,
                 semi2, semd2, seme2, semg2, sems2,
                 *, n_nodes, edges_per_worker, n_chunks, d):
    c = lax.axis_index("c")
    s = lax.axis_index("s")
    wid = s * NC + c
    nvec = d // 16
    zrows = zbuf.shape[0]
    n_row_chunks = n_nodes // zrows
    tail = n_nodes - n_row_chunks * zrows
    per_sub = (n_row_chunks + NS - 1) // NS

    def zero_row(i, carry):
        for j in range(nvec):
            zbuf[i, pl.ds(j * 16, 16)] = jnp.zeros((16,), jnp.float32)
        return carry

    lax.fori_loop(0, zrows, zero_row, 0)
    for k in range(per_sub):
        cid = k * NS + s

        @pl.when(cid < n_row_chunks)
        def _():
            pltpu.sync_copy(zbuf, agg_sh.at[pl.ds(cid * zrows, zrows)])
    if tail:
        @pl.when(s == NS - 1)
        def _():
            pltpu.sync_copy(zbuf.at[pl.ds(0, tail)],
                            agg_sh.at[pl.ds(n_row_chunks * zrows, tail)])
    plsc.subcore_barrier()

    base0 = wid * edges_per_worker

    bufs = [
        dict(src=src0, dst=dst0, buf=m0,
             semi=semi0, semd=semd0, seme=seme0, semg=semg0, sems=sems0),
        dict(src=src1, dst=dst1, buf=m1,
             semi=semi1, semd=semd1, seme=seme1, semg=semg1, sems=sems1),
        dict(src=src2, dst=dst2, buf=m2,
             semi=semi2, semd=semd2, seme=seme2, semg=semg2, sems=sems2),
    ]

    def issue_loads(b, k):
        base = base0 + k * CHUNK
        pltpu.async_copy(src_hbm.at[pl.ds(base, CHUNK)], b["src"], b["semi"])
        pltpu.async_copy(dst_hbm.at[pl.ds(base, CHUNK)], b["dst"], b["semd"])
        pltpu.async_copy(e_hbm.at[pl.ds(base, CHUNK)], b["buf"], b["seme"])

    def wait_src(b, k):
        base = base0 + k * CHUNK
        pltpu.make_async_copy(src_hbm.at[pl.ds(base, CHUNK)], b["src"],
                              b["semi"]).wait()

    def wait_dst(b, k):
        base = base0 + k * CHUNK
        pltpu.make_async_copy(dst_hbm.at[pl.ds(base, CHUNK)], b["dst"],
                              b["semd"]).wait()

    def wait_e(b, k):
        base = base0 + k * CHUNK
        pltpu.make_async_copy(e_hbm.at[pl.ds(base, CHUNK)], b["buf"],
                              b["seme"]).wait()

    def issue_gather_add(b):
        pltpu.async_copy(h_hbm.at[b["src"]], b["buf"], b["semg"], add=True)

    def wait_gather_add(b):
        pltpu.make_async_copy(h_hbm.at[b["src"]], b["buf"], b["semg"]).wait()

    def issue_scatter(b):
        pltpu.async_copy(b["buf"], agg_sh.at[b["dst"]], b["sems"], add=True)

    def wait_scatter(b):
        pltpu.make_async_copy(b["buf"], agg_sh.at[b["dst"]], b["sems"]).wait()

    def compute(b):
        m_v = b["buf"]

        def row4(r4, inner):
            for u in range(4):
                r = r4 * 4 + u
                for j in range(nvec):
                    sl = pl.ds(j * 16, 16)
                    m_v[r, sl] = jnp.maximum(m_v[r, sl], 0.0)
            return inner

        lax.fori_loop(0, CHUNK // 4, row4, 0)

    def step(k, prv, cur, nxt):
        @pl.when(k >= 1)
        def _():
            wait_scatter(prv)

        @pl.when(k + 2 < n_chunks)
        def _():
            issue_loads(prv, k + 2)

        wait_gather_add(cur)

        @pl.when(k + 1 < n_chunks)
        def _():
            wait_src(nxt, k + 1)
            wait_e(nxt, k + 1)
            issue_gather_add(nxt)

        compute(cur)
        wait_dst(cur, k)
        issue_scatter(cur)

    issue_loads(bufs[0], 0)
    issue_loads(bufs[1], 1)
    wait_src(bufs[0], 0)
    wait_e(bufs[0], 0)
    issue_gather_add(bufs[0])
    step(0, bufs[2], bufs[0], bufs[1])
    step(1, bufs[0], bufs[1], bufs[2])

    def triple(t, carry):
        step(3 * t + 2, bufs[1], bufs[2], bufs[0])
        step(3 * t + 3, bufs[2], bufs[0], bufs[1])
        step(3 * t + 4, bufs[0], bufs[1], bufs[2])
        return carry

    lax.fori_loop(0, (n_chunks - 2) // 3, triple, 0)
    wait_scatter(bufs[(n_chunks - 1) % 3])
    plsc.subcore_barrier()

    for k in range(per_sub):
        cid = k * NS + s

        @pl.when(cid < n_row_chunks)
        def _():
            off = cid * zrows
            pltpu.sync_copy(agg_sh.at[pl.ds(off, zrows)],
                            out_hbm.at[c, pl.ds(off, zrows)])
    if tail:
        @pl.when(s == NS - 1)
        def _():
            off = n_row_chunks * zrows
            pltpu.sync_copy(agg_sh.at[pl.ds(off, tail)],
                            out_hbm.at[c, pl.ds(off, tail)])


def _sc_agg(h, e, src, dst):
    n_nodes, d = h.shape
    n_edges = e.shape[0]
    epw = n_edges // (NC * NS)
    n_chunks = epw // CHUNK
    assert n_chunks >= 2 and (n_chunks - 2) % 3 == 0
    zrows = 64
    body = functools.partial(
        _sc_agg_body, n_nodes=n_nodes, edges_per_worker=epw,
        n_chunks=n_chunks, d=d)
    mesh = plsc.VectorSubcoreMesh(core_axis_name="c", subcore_axis_name="s")
    f = pl.kernel(
        body,
        out_type=jax.ShapeDtypeStruct((NC, n_nodes, d), jnp.float32),
        mesh=mesh,
        scratch_types=(
            [pltpu.VMEM((CHUNK,), jnp.int32),
             pltpu.VMEM((CHUNK,), jnp.int32),
             pltpu.VMEM((CHUNK, d), jnp.float32)] * 3
            + [pltpu.VMEM((zrows, d), jnp.float32),
               pltpu.VMEM_SHARED((n_nodes, d), jnp.float32)]
            + [pltpu.SemaphoreType.DMA] * 15
        ),
    )
    return f(h, e, src, dst)


def kernel(x, edge_index, edge_attr, batch, Wenc, benc, Wedge, bedge,
           W1, b1, W2, b2, gamma, beta, Wh1, bh1, Wh2, bh2, Wh3, bh3):
    num_layers = Wedge.shape[0]
    num_groups = 64
    src = edge_index[0]
    dst = edge_index[1]
    h = _encode(x, Wenc, benc.reshape(1, -1))
    for i in range(num_layers):
        e = _edge_transform(edge_attr, Wedge[i], bedge[i].reshape(1, -1))
        parts = _sc_agg(h, e, src, dst)
        h = _post(h, parts[0], parts[1], W1[i], b1[i].reshape(1, -1),
                  W2[i], b2[i].reshape(1, -1), gamma[i].reshape(1, -1),
                  beta[i].reshape(1, -1))
    out = _pool_head(h, batch.reshape(-1, 1), Wh1, bh1.reshape(1, -1),
                     Wh2, bh2.reshape(1, -1), Wh3, bh3.reshape(1, -1),
                     num_groups)
    return out.reshape(-1)
